# Initial kernel scaffold; baseline (speedup 1.0000x reference)
import jax
import jax.numpy as jnp
from jax.experimental import pallas as pl


def kernel(x, experts, gate_W, gate_b):
    raise NotImplementedError("write your pallas kernel here")


# speedup vs baseline: 1.4583x; 1.4583x over previous
import functools

import jax
import jax.numpy as jnp
from jax import lax
from jax.experimental import pallas as pl
from jax.experimental.pallas import tpu as pltpu

D = 768
HID = 768
E = 8
TOPK = 2
TOK = 8192
_ACTS = ["gelu", "silu", "relu", "leaky_relu"]
_DEPTHS = [1, 2, 3]
_SCALES = [0.5, 1.0, 1.5]

BG = 1024
BT = 512


def _cfg_k(i):
    return _ACTS[i % 4], _DEPTHS[i % 3], int(HID * _SCALES[i % 3])


def _apply_act(name, h):
    if name == "gelu":
        return 0.5 * h * (1.0 + lax.erf(h * 0.7071067811865476))
    if name == "silu":
        return jax.nn.silu(h)
    if name == "relu":
        return jax.nn.relu(h)
    return jax.nn.leaky_relu(h, negative_slope=0.01)


def _layernorm(h, g, b):
    mu = jnp.mean(h, axis=-1, keepdims=True)
    var = jnp.mean((h - mu) * (h - mu), axis=-1, keepdims=True)
    return (h - mu) / jnp.sqrt(var + 1e-5) * g + b


def _gate_body(x_ref, gw_ref, gb_ref,
               wfull_ref, idx_ref, wtop_ref, pos_ref, counts_ref,
               cnt_ref):
    pid = pl.program_id(0)

    @pl.when(pid == 0)
    def _():
        cnt_ref[...] = jnp.zeros_like(cnt_ref)

    x = x_ref[...]
    logits = jnp.dot(x, gw_ref[...], preferred_element_type=jnp.float32)
    logits = logits + gb_ref[...]
    m = jnp.max(logits, axis=-1, keepdims=True)
    p = jnp.exp(logits - m)
    z = jnp.sum(p, axis=-1, keepdims=True)
    v = p / z

    col = lax.broadcasted_iota(jnp.int32, (BG, E), 1)
    v1 = jnp.max(v, axis=-1, keepdims=True)
    e1 = jnp.min(jnp.where(v == v1, col, E), axis=-1, keepdims=True)
    oh1 = col == e1
    v_m = jnp.where(oh1, -jnp.inf, v)
    v2 = jnp.max(v_m, axis=-1, keepdims=True)
    e2 = jnp.min(jnp.where(v_m == v2, col, E), axis=-1, keepdims=True)
    oh2 = col == e2

    denom = v1 + v2 + 1e-9
    maskf = (oh1 | oh2).astype(jnp.float32)
    wfull_ref[...] = maskf * v / denom
    wtop_ref[...] = jnp.concatenate([v1 / denom, v2 / denom], axis=1)
    idx_ref[...] = jnp.concatenate([e1, e2], axis=1)

    r_i = lax.broadcasted_iota(jnp.int32, (BG, BG), 0)
    c_i = lax.broadcasted_iota(jnp.int32, (BG, BG), 1)
    tril = (r_i >= c_i).astype(jnp.float32)
    cum = jnp.dot(tril, maskf, preferred_element_type=jnp.float32)
    cnt = cnt_ref[...]
    pos_x = cnt + cum - maskf
    p1 = jnp.sum(jnp.where(oh1, pos_x, 0.0), axis=-1, keepdims=True)
    p2 = jnp.sum(jnp.where(oh2, pos_x, 0.0), axis=-1, keepdims=True)
    pos_ref[...] = jnp.concatenate([p1, p2], axis=1).astype(jnp.int32)
    new_cnt = cnt + cum[BG - 1:BG, :]
    cnt_ref[...] = new_cnt
    counts_ref[...] = new_cnt.astype(jnp.int32)


def _gate(x, gate_W, gate_b):
    grid = TOK // BG
    out_shapes = (
        jax.ShapeDtypeStruct((TOK, E), jnp.float32),
        jax.ShapeDtypeStruct((TOK, TOPK), jnp.int32),
        jax.ShapeDtypeStruct((TOK, TOPK), jnp.float32),
        jax.ShapeDtypeStruct((TOK, TOPK), jnp.int32),
        jax.ShapeDtypeStruct((1, E), jnp.int32),
    )
    return pl.pallas_call(
        _gate_body,
        grid=(grid,),
        in_specs=[
            pl.BlockSpec((BG, D), lambda i: (i, 0)),
            pl.BlockSpec((D, E), lambda i: (0, 0)),
            pl.BlockSpec((1, E), lambda i: (0, 0)),
        ],
        out_specs=(
            pl.BlockSpec((BG, E), lambda i: (i, 0)),
            pl.BlockSpec((BG, TOPK), lambda i: (i, 0)),
            pl.BlockSpec((BG, TOPK), lambda i: (i, 0)),
            pl.BlockSpec((BG, TOPK), lambda i: (i, 0)),
            pl.BlockSpec((1, E), lambda i: (0, 0)),
        ),
        out_shape=out_shapes,
        scratch_shapes=[pltpu.VMEM((1, E), jnp.float32)],
    )(x, gate_W, gate_b.reshape(1, E))


def _mlp_body(act, depth, hid, x_ref, *refs):
    out_ref = refs[-1]
    h = x_ref[...]
    for j in range(depth):
        W, b, g, beta = refs[4 * j: 4 * j + 4]
        h = jnp.dot(h, W[...], preferred_element_type=jnp.float32) + b[...]
        h = _apply_act(act, h)
        h = _layernorm(h, g[...], beta[...])
    Wo, bo, go, betao = refs[4 * depth: 4 * depth + 4]
    h = jnp.dot(h, Wo[...], preferred_element_type=jnp.float32) + bo[...]
    h = _layernorm(h, go[...], betao[...])
    out_ref[...] = h


def _expert_dense(i, ep, x):
    act, depth, hid = _cfg_k(i)
    grid = TOK // BT
    args = [x]
    in_specs = [pl.BlockSpec((BT, D), lambda i: (i, 0))]
    in_dim = D
    for lyr in ep["layers"]:
        args += [lyr["W"], lyr["b"].reshape(1, hid), lyr["g"].reshape(1, hid),
                 lyr["beta"].reshape(1, hid)]
        in_specs += [
            pl.BlockSpec((in_dim, hid), lambda i: (0, 0)),
            pl.BlockSpec((1, hid), lambda i: (0, 0)),
            pl.BlockSpec((1, hid), lambda i: (0, 0)),
            pl.BlockSpec((1, hid), lambda i: (0, 0)),
        ]
        in_dim = hid
    o = ep["out"]
    args += [o["W"], o["b"].reshape(1, D), o["g"].reshape(1, D),
             o["beta"].reshape(1, D)]
    in_specs += [
        pl.BlockSpec((in_dim, D), lambda i: (0, 0)),
        pl.BlockSpec((1, D), lambda i: (0, 0)),
        pl.BlockSpec((1, D), lambda i: (0, 0)),
        pl.BlockSpec((1, D), lambda i: (0, 0)),
    ]
    return pl.pallas_call(
        functools.partial(_mlp_body, act, depth, hid),
        grid=(grid,),
        in_specs=in_specs,
        out_specs=pl.BlockSpec((BT, D), lambda i: (i, 0)),
        out_shape=jax.ShapeDtypeStruct((TOK, D), jnp.float32),
    )(*args)


def _combine_body(w_ref, *refs):
    out_ref = refs[-1]
    w = w_ref[...]
    acc = jnp.zeros_like(out_ref)
    for e in range(E):
        acc = acc + refs[e][...] * w[:, e:e + 1]
    out_ref[...] = acc


def _combine(wfull, ys):
    grid = TOK // BT
    in_specs = [pl.BlockSpec((BT, E), lambda i: (i, 0))]
    in_specs += [pl.BlockSpec((BT, D), lambda i: (i, 0)) for _ in range(E)]
    return pl.pallas_call(
        _combine_body,
        grid=(grid,),
        in_specs=in_specs,
        out_specs=pl.BlockSpec((BT, D), lambda i: (i, 0)),
        out_shape=jax.ShapeDtypeStruct((TOK, D), jnp.float32),
    )(wfull, *ys)


def kernel(x, experts, gate_W, gate_b):
    wfull, idx, wtop, pos, counts = _gate(x, gate_W, gate_b)
    ys = [_expert_dense(i, ep, x) for i, ep in enumerate(experts)]
    return _combine(wfull, ys)


# speedup vs baseline: 1.7081x; 1.1713x over previous
import functools

import jax
import jax.numpy as jnp
from jax import lax
from jax.experimental import pallas as pl
from jax.experimental.pallas import tpu as pltpu
from jax.experimental.pallas import tpu_sc as plsc

D = 768
HID = 768
E = 8
TOPK = 2
TOK = 8192
_ACTS = ["gelu", "silu", "relu", "leaky_relu"]
_DEPTHS = [1, 2, 3]
_SCALES = [0.5, 1.0, 1.5]

BG = 1024
BT = 512
ECAP = TOK
NBLK_MAX = ECAP // BT
CH = 128
CPB = BT // CH
NW = 32


def _cfg_k(i):
    return _ACTS[i % 4], _DEPTHS[i % 3], int(HID * _SCALES[i % 3])


def _apply_act(name, h):
    if name == "gelu":
        return 0.5 * h * (1.0 + lax.erf(h * 0.7071067811865476))
    if name == "silu":
        return jax.nn.silu(h)
    if name == "relu":
        return jax.nn.relu(h)
    return jax.nn.leaky_relu(h, negative_slope=0.01)


def _layernorm(h, g, b):
    mu = jnp.mean(h, axis=-1, keepdims=True)
    var = jnp.mean((h - mu) * (h - mu), axis=-1, keepdims=True)
    return (h - mu) / jnp.sqrt(var + 1e-5) * g + b


def _gate_body(x_ref, gw_ref, gb_ref,
               idx_ref, wtop_ref, pos_ref, counts_ref,
               cnt_ref):
    pid = pl.program_id(0)

    @pl.when(pid == 0)
    def _():
        cnt_ref[...] = jnp.zeros_like(cnt_ref)

    x = x_ref[...]
    logits = jnp.dot(x, gw_ref[...], preferred_element_type=jnp.float32)
    logits = logits + gb_ref[...]
    m = jnp.max(logits, axis=-1, keepdims=True)
    p = jnp.exp(logits - m)
    z = jnp.sum(p, axis=-1, keepdims=True)
    v = p / z

    col = lax.broadcasted_iota(jnp.int32, (BG, E), 1)
    v1 = jnp.max(v, axis=-1, keepdims=True)
    e1 = jnp.min(jnp.where(v == v1, col, E), axis=-1, keepdims=True)
    oh1 = col == e1
    v_m = jnp.where(oh1, -jnp.inf, v)
    v2 = jnp.max(v_m, axis=-1, keepdims=True)
    e2 = jnp.min(jnp.where(v_m == v2, col, E), axis=-1, keepdims=True)
    oh2 = col == e2

    denom = v1 + v2 + 1e-9
    maskf = (oh1 | oh2).astype(jnp.float32)
    wtop_ref[...] = jnp.concatenate([v1 / denom, v2 / denom], axis=1)
    idx_ref[...] = jnp.concatenate([e1, e2], axis=1)

    r_i = lax.broadcasted_iota(jnp.int32, (BG, BG), 0)
    c_i = lax.broadcasted_iota(jnp.int32, (BG, BG), 1)
    tril = (r_i >= c_i).astype(jnp.float32)
    cum = jnp.dot(tril, maskf, preferred_element_type=jnp.float32)
    cnt = cnt_ref[...]
    pos_x = cnt + cum - maskf
    p1 = jnp.sum(jnp.where(oh1, pos_x, 0.0), axis=-1, keepdims=True)
    p2 = jnp.sum(jnp.where(oh2, pos_x, 0.0), axis=-1, keepdims=True)
    pos_ref[...] = jnp.concatenate([p1, p2], axis=1).astype(jnp.int32)
    new_cnt = cnt + cum[BG - 1:BG, :]
    cnt_ref[...] = new_cnt
    counts_ref[...] = new_cnt.astype(jnp.int32)


def _gate(x, gate_W, gate_b):
    grid = TOK // BG
    out_shapes = (
        jax.ShapeDtypeStruct((TOK, TOPK), jnp.int32),
        jax.ShapeDtypeStruct((TOK, TOPK), jnp.float32),
        jax.ShapeDtypeStruct((TOK, TOPK), jnp.int32),
        jax.ShapeDtypeStruct((1, E), jnp.int32),
    )
    return pl.pallas_call(
        _gate_body,
        grid=(grid,),
        in_specs=[
            pl.BlockSpec((BG, D), lambda i: (i, 0)),
            pl.BlockSpec((D, E), lambda i: (0, 0)),
            pl.BlockSpec((1, E), lambda i: (0, 0)),
        ],
        out_specs=(
            pl.BlockSpec((BG, TOPK), lambda i: (i, 0)),
            pl.BlockSpec((BG, TOPK), lambda i: (i, 0)),
            pl.BlockSpec((BG, TOPK), lambda i: (i, 0)),
            pl.BlockSpec((1, E), lambda i: (0, 0)),
        ),
        out_shape=out_shapes,
        scratch_shapes=[pltpu.VMEM((1, E), jnp.float32)],
    )(x, gate_W, gate_b.reshape(1, E))


def _sc_gather(x, sorted_tok, nchunk):
    mesh = plsc.VectorSubcoreMesh(core_axis_name="c", subcore_axis_name="s")

    @functools.partial(
        pl.kernel, mesh=mesh,
        out_type=tuple(jax.ShapeDtypeStruct((ECAP, D), jnp.float32)
                       for _ in range(E)),
        scratch_types=[
            pltpu.VMEM((CH,), jnp.int32),
            pltpu.VMEM((CH, D), jnp.float32),
            pltpu.VMEM((16,), jnp.int32),
            pltpu.SemaphoreType.DMA,
        ],
    )
    def k(x_hbm, st_hbm, nch_hbm, *rest):
        xg_hbms = rest[:E]
        idx_v, rows_v, nch_v, sem = rest[E:]
        wid = lax.axis_index("s") * 2 + lax.axis_index("c")
        pltpu.sync_copy(nch_hbm, nch_v)
        nch_vec = nch_v[...]
        for e in range(E):
            n = nch_vec[e]
            kmax = (n - wid + NW - 1) // NW

            def body(kk, _, e=e):
                c = wid + kk * NW
                base = c * CH
                pltpu.sync_copy(st_hbm.at[pl.ds(e * ECAP + base, CH)], idx_v)
                pltpu.async_copy(x_hbm.at[idx_v], rows_v, sem).wait()
                pltpu.sync_copy(rows_v, xg_hbms[e].at[pl.ds(base, CH)])
                return 0

            lax.fori_loop(0, kmax, body, 0)

    return k(x, sorted_tok, nchunk)


def _sc_scatter(ygs, dest, nchunk):
    mesh = plsc.VectorSubcoreMesh(core_axis_name="c", subcore_axis_name="s")

    @functools.partial(
        pl.kernel, mesh=mesh,
        out_type=jax.ShapeDtypeStruct((TOPK * TOK + CH, D), jnp.float32),
        scratch_types=[
            pltpu.VMEM((CH,), jnp.int32),
            pltpu.VMEM((CH, D), jnp.float32),
            pltpu.VMEM((16,), jnp.int32),
            pltpu.SemaphoreType.DMA,
        ],
    )
    def k(*args):
        yg_hbms = args[:E]
        dest_hbm, nch_hbm, yc_hbm, idx_v, rows_v, nch_v, sem = args[E:]
        wid = lax.axis_index("s") * 2 + lax.axis_index("c")
        pltpu.sync_copy(nch_hbm, nch_v)
        nch_vec = nch_v[...]
        for e in range(E):
            n = nch_vec[e]
            kmax = (n - wid + NW - 1) // NW

            def body(kk, _, e=e):
                c = wid + kk * NW
                base = c * CH
                pltpu.sync_copy(dest_hbm.at[pl.ds(e * ECAP + base, CH)], idx_v)
                pltpu.sync_copy(yg_hbms[e].at[pl.ds(base, CH)], rows_v)
                pltpu.async_copy(rows_v, yc_hbm.at[idx_v], sem).wait()
                return 0

            lax.fori_loop(0, kmax, body, 0)

    return k(*ygs, dest, nchunk)


def _mlp_body(act, depth, nrefs, s_ref, *refs):
    out_ref = refs[-1]
    i = pl.program_id(0)

    @pl.when(i < s_ref[0])
    def _():
        h = refs[0][...]
        for j in range(depth):
            W, b, g, beta = refs[1 + 4 * j: 5 + 4 * j]
            h = jnp.dot(h, W[...], preferred_element_type=jnp.float32) + b[...]
            h = _apply_act(act, h)
            h = _layernorm(h, g[...], beta[...])
        Wo, bo, go, betao = refs[1 + 4 * depth: 5 + 4 * depth]
        h = jnp.dot(h, Wo[...], preferred_element_type=jnp.float32) + bo[...]
        h = _layernorm(h, go[...], betao[...])
        out_ref[...] = h * refs[-2][...]


def _expert_sparse(i, ep, xg_e, wg_e, nblk_e):
    act, depth, hid = _cfg_k(i)

    def xmap(b, s):
        j = jnp.maximum(jnp.minimum(b, s[0] - 1), 0)
        return (j, 0)

    def cmap(b, s):
        return (0, 0)

    args = [nblk_e, xg_e]
    in_specs = [pl.BlockSpec((BT, D), xmap)]
    in_dim = D
    for lyr in ep["layers"]:
        args += [lyr["W"], lyr["b"].reshape(1, hid), lyr["g"].reshape(1, hid),
                 lyr["beta"].reshape(1, hid)]
        in_specs += [
            pl.BlockSpec((in_dim, hid), cmap),
            pl.BlockSpec((1, hid), cmap),
            pl.BlockSpec((1, hid), cmap),
            pl.BlockSpec((1, hid), cmap),
        ]
        in_dim = hid
    o = ep["out"]
    args += [o["W"], o["b"].reshape(1, D), o["g"].reshape(1, D),
             o["beta"].reshape(1, D)]
    in_specs += [
        pl.BlockSpec((in_dim, D), cmap),
        pl.BlockSpec((1, D), cmap),
        pl.BlockSpec((1, D), cmap),
        pl.BlockSpec((1, D), cmap),
    ]
    args += [wg_e]
    in_specs += [pl.BlockSpec((BT, 1), xmap)]
    return pl.pallas_call(
        functools.partial(_mlp_body, act, depth, len(in_specs)),
        grid_spec=pltpu.PrefetchScalarGridSpec(
            num_scalar_prefetch=1,
            grid=(NBLK_MAX,),
            in_specs=in_specs,
            out_specs=pl.BlockSpec((BT, D), xmap),
        ),
        out_shape=jax.ShapeDtypeStruct((ECAP, D), jnp.float32),
    )(*args)


def _add_body(a_ref, b_ref, o_ref):
    o_ref[...] = a_ref[...] + b_ref[...]


def _final_add(ycat):
    grid = TOK // BT
    return pl.pallas_call(
        _add_body,
        grid=(grid,),
        in_specs=[
            pl.BlockSpec((BT, D), lambda i: (i, 0)),
            pl.BlockSpec((BT, D), lambda i: (TOK // BT + i, 0)),
        ],
        out_specs=pl.BlockSpec((BT, D), lambda i: (i, 0)),
        out_shape=jax.ShapeDtypeStruct((TOK, D), jnp.float32),
    )(ycat, ycat)


def kernel(x, experts, gate_W, gate_b):
    idx, wtop, pos, counts = _gate(x, gate_W, gate_b)

    tok = jnp.arange(TOK, dtype=jnp.int32)
    slot = idx * ECAP + pos
    s0, s1 = slot[:, 0], slot[:, 1]
    sorted_tok = (jnp.zeros((E * ECAP,), jnp.int32)
                  .at[s0].set(tok, unique_indices=True)
                  .at[s1].set(tok, unique_indices=True))
    wg = (jnp.zeros((E * ECAP,), jnp.float32)
          .at[s0].set(wtop[:, 0], unique_indices=True)
          .at[s1].set(wtop[:, 1], unique_indices=True)).reshape(E * ECAP, 1)
    dest = (jnp.full((E * ECAP,), TOPK * TOK, jnp.int32)
            .at[s0].set(tok, unique_indices=True)
            .at[s1].set(tok + TOK, unique_indices=True))
    nblk = (counts[0] + BT - 1) // BT
    nchunk16 = jnp.pad(nblk * CPB, (0, 16 - E))

    xgs = _sc_gather(x, sorted_tok, nchunk16)
    ygs = [
        _expert_sparse(i, ep, xgs[i], wg[i * ECAP:(i + 1) * ECAP], nblk[i:i + 1])
        for i, ep in enumerate(experts)
    ]
    ycat = _sc_scatter(ygs, dest, nchunk16)
    return _final_add(ycat)


# speedup vs baseline: 1.7272x; 1.0112x over previous
import functools

import jax
import jax.numpy as jnp
from jax import lax
from jax.experimental import pallas as pl
from jax.experimental.pallas import tpu as pltpu
from jax.experimental.pallas import tpu_sc as plsc

D = 768
HID = 768
E = 8
TOPK = 2
TOK = 8192
_ACTS = ["gelu", "silu", "relu", "leaky_relu"]
_DEPTHS = [1, 2, 3]
_SCALES = [0.5, 1.0, 1.5]

BG = 1024
BT = 512
ECAP = TOK
NBLK_MAX = ECAP // BT
CH = 64
CPB = BT // CH
NW = 32


def _cfg_k(i):
    return _ACTS[i % 4], _DEPTHS[i % 3], int(HID * _SCALES[i % 3])


def _apply_act(name, h):
    if name == "gelu":
        return 0.5 * h * (1.0 + lax.erf(h * 0.7071067811865476))
    if name == "silu":
        return jax.nn.silu(h)
    if name == "relu":
        return jax.nn.relu(h)
    return jax.nn.leaky_relu(h, negative_slope=0.01)


def _layernorm(h, g, b):
    mu = jnp.mean(h, axis=-1, keepdims=True)
    var = jnp.mean((h - mu) * (h - mu), axis=-1, keepdims=True)
    return (h - mu) / jnp.sqrt(var + 1e-5) * g + b


def _gate_body(x_ref, gw_ref, gb_ref,
               idx_ref, wtop_ref, pos_ref, counts_ref,
               cnt_ref):
    pid = pl.program_id(0)

    @pl.when(pid == 0)
    def _():
        cnt_ref[...] = jnp.zeros_like(cnt_ref)

    x = x_ref[...]
    logits = jnp.dot(x, gw_ref[...], preferred_element_type=jnp.float32)
    logits = logits + gb_ref[...]
    m = jnp.max(logits, axis=-1, keepdims=True)
    p = jnp.exp(logits - m)
    z = jnp.sum(p, axis=-1, keepdims=True)
    v = p / z

    col = lax.broadcasted_iota(jnp.int32, (BG, E), 1)
    v1 = jnp.max(v, axis=-1, keepdims=True)
    e1 = jnp.min(jnp.where(v == v1, col, E), axis=-1, keepdims=True)
    oh1 = col == e1
    v_m = jnp.where(oh1, -jnp.inf, v)
    v2 = jnp.max(v_m, axis=-1, keepdims=True)
    e2 = jnp.min(jnp.where(v_m == v2, col, E), axis=-1, keepdims=True)
    oh2 = col == e2

    denom = v1 + v2 + 1e-9
    maskf = (oh1 | oh2).astype(jnp.float32)
    wtop_ref[...] = jnp.concatenate([v1 / denom, v2 / denom], axis=1)
    idx_ref[...] = jnp.concatenate([e1, e2], axis=1)

    r_i = lax.broadcasted_iota(jnp.int32, (BG, BG), 0)
    c_i = lax.broadcasted_iota(jnp.int32, (BG, BG), 1)
    tril = (r_i >= c_i).astype(jnp.float32)
    cum = jnp.dot(tril, maskf, preferred_element_type=jnp.float32)
    cnt = cnt_ref[...]
    pos_x = cnt + cum - maskf
    p1 = jnp.sum(jnp.where(oh1, pos_x, 0.0), axis=-1, keepdims=True)
    p2 = jnp.sum(jnp.where(oh2, pos_x, 0.0), axis=-1, keepdims=True)
    pos_ref[...] = jnp.concatenate([p1, p2], axis=1).astype(jnp.int32)
    new_cnt = cnt + cum[BG - 1:BG, :]
    cnt_ref[...] = new_cnt
    counts_ref[...] = new_cnt.astype(jnp.int32)


def _gate(x, gate_W, gate_b):
    grid = TOK // BG
    out_shapes = (
        jax.ShapeDtypeStruct((TOK, TOPK), jnp.int32),
        jax.ShapeDtypeStruct((TOK, TOPK), jnp.float32),
        jax.ShapeDtypeStruct((TOK, TOPK), jnp.int32),
        jax.ShapeDtypeStruct((1, E), jnp.int32),
    )
    return pl.pallas_call(
        _gate_body,
        grid=(grid,),
        in_specs=[
            pl.BlockSpec((BG, D), lambda i: (i, 0)),
            pl.BlockSpec((D, E), lambda i: (0, 0)),
            pl.BlockSpec((1, E), lambda i: (0, 0)),
        ],
        out_specs=(
            pl.BlockSpec((BG, TOPK), lambda i: (i, 0)),
            pl.BlockSpec((BG, TOPK), lambda i: (i, 0)),
            pl.BlockSpec((BG, TOPK), lambda i: (i, 0)),
            pl.BlockSpec((1, E), lambda i: (0, 0)),
        ),
        out_shape=out_shapes,
        scratch_shapes=[pltpu.VMEM((1, E), jnp.float32)],
    )(x, gate_W, gate_b.reshape(1, E))


_SC_SCRATCH = [
    pltpu.VMEM((CH,), jnp.int32),
    pltpu.VMEM((CH,), jnp.int32),
    pltpu.VMEM((CH, D), jnp.float32),
    pltpu.VMEM((CH, D), jnp.float32),
    pltpu.VMEM((16,), jnp.int32),
    pltpu.SemaphoreType.DMA,
    pltpu.SemaphoreType.DMA,
    pltpu.SemaphoreType.DMA,
    pltpu.SemaphoreType.DMA,
]


def _sc_copy_loop(wid, nch_vec, scr, chunk_fn, drain_fn):
    idx0, idx1, rows0, rows1 = scr[0], scr[1], scr[2], scr[3]
    sa0, sa1, sb0, sb1 = scr[5], scr[6], scr[7], scr[8]
    bufs = ((idx0, rows0, sa0, sb0), (idx1, rows1, sa1, sb1))
    for e in range(E):
        n = nch_vec[e]
        kw = (n - wid + NW - 1) // NW

        def body(k2, _, e=e):
            for b in range(2):
                idxv, rowsv, semA, semB = bufs[b]
                kchunk = 2 * k2 + b

                @pl.when(kchunk < kw)
                def _(b=b, kchunk=kchunk, idxv=idxv, rowsv=rowsv,
                      semA=semA, semB=semB):
                    @pl.when(kchunk >= 2)
                    def _():
                        drain_fn(e, rowsv, idxv, semB)

                    c = wid + kchunk * NW
                    chunk_fn(e, c, idxv, rowsv, semA, semB)
            return 0

        lax.fori_loop(0, (kw + 1) // 2, body, 0)
        for b in range(2):
            idxv, rowsv, _, semB = bufs[b]

            @pl.when(kw >= b + 1)
            def _(e=e, rowsv=rowsv, idxv=idxv, semB=semB):
                drain_fn(e, rowsv, idxv, semB)


def _sc_gather(x, sorted_tok, nchunk):
    mesh = plsc.VectorSubcoreMesh(core_axis_name="c", subcore_axis_name="s")

    @functools.partial(
        pl.kernel, mesh=mesh,
        out_type=tuple(jax.ShapeDtypeStruct((ECAP, D), jnp.float32)
                       for _ in range(E)),
        scratch_types=list(_SC_SCRATCH),
    )
    def k(x_hbm, st_hbm, nch_hbm, *rest):
        xg_hbms = rest[:E]
        scr = rest[E:]
        nch_v = scr[4]
        wid = lax.axis_index("s") * 2 + lax.axis_index("c")
        pltpu.sync_copy(nch_hbm, nch_v)
        nch_vec = nch_v[...]

        def chunk_fn(e, c, idxv, rowsv, semA, semB):
            base = c * CH
            pltpu.sync_copy(st_hbm.at[pl.ds(e * ECAP + base, CH)], idxv)
            pltpu.async_copy(x_hbm.at[idxv], rowsv, semA).wait()
            pltpu.async_copy(rowsv, xg_hbms[e].at[pl.ds(base, CH)], semB)

        def drain_fn(e, rowsv, idxv, semB):
            pltpu.make_async_copy(
                rowsv, xg_hbms[e].at[pl.ds(0, CH)], semB).wait()

        _sc_copy_loop(wid, nch_vec, scr, chunk_fn, drain_fn)

    return k(x, sorted_tok, nchunk)


def _sc_scatter(ygs, dest, nchunk):
    mesh = plsc.VectorSubcoreMesh(core_axis_name="c", subcore_axis_name="s")

    @functools.partial(
        pl.kernel, mesh=mesh,
        out_type=jax.ShapeDtypeStruct((TOPK * TOK + CH, D), jnp.float32),
        scratch_types=list(_SC_SCRATCH),
    )
    def k(*args):
        yg_hbms = args[:E]
        dest_hbm, nch_hbm, yc_hbm = args[E:E + 3]
        scr = args[E + 3:]
        nch_v = scr[4]
        wid = lax.axis_index("s") * 2 + lax.axis_index("c")
        pltpu.sync_copy(nch_hbm, nch_v)
        nch_vec = nch_v[...]

        def chunk_fn(e, c, idxv, rowsv, semA, semB):
            base = c * CH
            pltpu.sync_copy(dest_hbm.at[pl.ds(e * ECAP + base, CH)], idxv)
            pltpu.async_copy(yg_hbms[e].at[pl.ds(base, CH)], rowsv, semA).wait()
            pltpu.async_copy(rowsv, yc_hbm.at[idxv], semB)

        def drain_fn(e, rowsv, idxv, semB):
            pltpu.make_async_copy(rowsv, yc_hbm.at[idxv], semB).wait()

        _sc_copy_loop(wid, nch_vec, scr, chunk_fn, drain_fn)

    return k(*ygs, dest, nchunk)


def _mlp_body(act, depth, nrefs, s_ref, *refs):
    out_ref = refs[-1]
    i = pl.program_id(0)

    @pl.when(i < s_ref[0])
    def _():
        h = refs[0][...]
        for j in range(depth):
            W, b, g, beta = refs[1 + 4 * j: 5 + 4 * j]
            h = jnp.dot(h, W[...], preferred_element_type=jnp.float32) + b[...]
            h = _apply_act(act, h)
            h = _layernorm(h, g[...], beta[...])
        Wo, bo, go, betao = refs[1 + 4 * depth: 5 + 4 * depth]
        h = jnp.dot(h, Wo[...], preferred_element_type=jnp.float32) + bo[...]
        h = _layernorm(h, go[...], betao[...])
        out_ref[...] = h * refs[-2][...]


def _expert_sparse(i, ep, xg_e, wg_e, nblk_e):
    act, depth, hid = _cfg_k(i)

    def xmap(b, s):
        j = jnp.maximum(jnp.minimum(b, s[0] - 1), 0)
        return (j, 0)

    def cmap(b, s):
        return (0, 0)

    args = [nblk_e, xg_e]
    in_specs = [pl.BlockSpec((BT, D), xmap)]
    in_dim = D
    for lyr in ep["layers"]:
        args += [lyr["W"], lyr["b"].reshape(1, hid), lyr["g"].reshape(1, hid),
                 lyr["beta"].reshape(1, hid)]
        in_specs += [
            pl.BlockSpec((in_dim, hid), cmap),
            pl.BlockSpec((1, hid), cmap),
            pl.BlockSpec((1, hid), cmap),
            pl.BlockSpec((1, hid), cmap),
        ]
        in_dim = hid
    o = ep["out"]
    args += [o["W"], o["b"].reshape(1, D), o["g"].reshape(1, D),
             o["beta"].reshape(1, D)]
    in_specs += [
        pl.BlockSpec((in_dim, D), cmap),
        pl.BlockSpec((1, D), cmap),
        pl.BlockSpec((1, D), cmap),
        pl.BlockSpec((1, D), cmap),
    ]
    args += [wg_e]
    in_specs += [pl.BlockSpec((BT, 1), xmap)]
    return pl.pallas_call(
        functools.partial(_mlp_body, act, depth, len(in_specs)),
        grid_spec=pltpu.PrefetchScalarGridSpec(
            num_scalar_prefetch=1,
            grid=(NBLK_MAX,),
            in_specs=in_specs,
            out_specs=pl.BlockSpec((BT, D), xmap),
        ),
        out_shape=jax.ShapeDtypeStruct((ECAP, D), jnp.float32),
    )(*args)


def _add_body(a_ref, b_ref, o_ref):
    o_ref[...] = a_ref[...] + b_ref[...]


def _final_add(ycat):
    grid = TOK // BT
    return pl.pallas_call(
        _add_body,
        grid=(grid,),
        in_specs=[
            pl.BlockSpec((BT, D), lambda i: (i, 0)),
            pl.BlockSpec((BT, D), lambda i: (TOK // BT + i, 0)),
        ],
        out_specs=pl.BlockSpec((BT, D), lambda i: (i, 0)),
        out_shape=jax.ShapeDtypeStruct((TOK, D), jnp.float32),
    )(ycat, ycat)


def kernel(x, experts, gate_W, gate_b):
    idx, wtop, pos, counts = _gate(x, gate_W, gate_b)

    tok = jnp.arange(TOK, dtype=jnp.int32)
    slot = idx * ECAP + pos
    s0, s1 = slot[:, 0], slot[:, 1]
    sorted_tok = (jnp.zeros((E * ECAP,), jnp.int32)
                  .at[s0].set(tok, unique_indices=True)
                  .at[s1].set(tok, unique_indices=True))
    wg = (jnp.zeros((E * ECAP,), jnp.float32)
          .at[s0].set(wtop[:, 0], unique_indices=True)
          .at[s1].set(wtop[:, 1], unique_indices=True)).reshape(E * ECAP, 1)
    dest = (jnp.full((E * ECAP,), TOPK * TOK, jnp.int32)
            .at[s0].set(tok, unique_indices=True)
            .at[s1].set(tok + TOK, unique_indices=True))
    nblk = (counts[0] + BT - 1) // BT
    nchunk16 = jnp.pad(nblk * CPB, (0, 16 - E))

    xgs = _sc_gather(x, sorted_tok, nchunk16)
    ygs = [
        _expert_sparse(i, ep, xgs[i], wg[i * ECAP:(i + 1) * ECAP], nblk[i:i + 1])
        for i, ep in enumerate(experts)
    ]
    ycat = _sc_scatter(ygs, dest, nchunk16)
    return _final_add(ycat)


# speedup vs baseline: 1.8693x; 1.0822x over previous
import functools

import jax
import jax.numpy as jnp
from jax import lax
from jax.experimental import pallas as pl
from jax.experimental.pallas import tpu as pltpu
from jax.experimental.pallas import tpu_sc as plsc

D = 768
HID = 768
E = 8
TOPK = 2
TOK = 8192
_ACTS = ["gelu", "silu", "relu", "leaky_relu"]
_DEPTHS = [1, 2, 3]
_SCALES = [0.5, 1.0, 1.5]

BG = 1024
BT = 512
ECAP = TOK
NBLK_MAX = ECAP // BT
CH = 64
CPB = BT // CH
NW = 32


def _cfg_k(i):
    return _ACTS[i % 4], _DEPTHS[i % 3], int(HID * _SCALES[i % 3])


def _apply_act(name, h):
    if name == "gelu":
        return 0.5 * h * (1.0 + lax.erf(h * 0.7071067811865476))
    if name == "silu":
        return jax.nn.silu(h)
    if name == "relu":
        return jax.nn.relu(h)
    return jax.nn.leaky_relu(h, negative_slope=0.01)


def _layernorm(h, g, b):
    mu = jnp.mean(h, axis=-1, keepdims=True)
    var = jnp.mean((h - mu) * (h - mu), axis=-1, keepdims=True)
    return (h - mu) / jnp.sqrt(var + 1e-5) * g + b


def _gate_body(x_ref, gw_ref, gb_ref,
               idx_ref, wtop_ref, pos_ref, counts_ref,
               cnt_ref):
    pid = pl.program_id(0)

    @pl.when(pid == 0)
    def _():
        cnt_ref[...] = jnp.zeros_like(cnt_ref)

    x = x_ref[...]
    logits = jnp.dot(x, gw_ref[...], preferred_element_type=jnp.float32)
    logits = logits + gb_ref[...]
    m = jnp.max(logits, axis=-1, keepdims=True)
    p = jnp.exp(logits - m)
    z = jnp.sum(p, axis=-1, keepdims=True)
    v = p / z

    col = lax.broadcasted_iota(jnp.int32, (BG, E), 1)
    v1 = jnp.max(v, axis=-1, keepdims=True)
    e1 = jnp.min(jnp.where(v == v1, col, E), axis=-1, keepdims=True)
    oh1 = col == e1
    v_m = jnp.where(oh1, -jnp.inf, v)
    v2 = jnp.max(v_m, axis=-1, keepdims=True)
    e2 = jnp.min(jnp.where(v_m == v2, col, E), axis=-1, keepdims=True)
    oh2 = col == e2

    denom = v1 + v2 + 1e-9
    maskf = (oh1 | oh2).astype(jnp.float32)
    wtop_ref[...] = jnp.concatenate([v1 / denom, v2 / denom], axis=1)
    idx_ref[...] = jnp.concatenate([e1, e2], axis=1)

    r_i = lax.broadcasted_iota(jnp.int32, (BG, BG), 0)
    c_i = lax.broadcasted_iota(jnp.int32, (BG, BG), 1)
    tril = (r_i >= c_i).astype(jnp.float32)
    cum = jnp.dot(tril, maskf, preferred_element_type=jnp.float32)
    cnt = cnt_ref[...]
    pos_x = cnt + cum - maskf
    p1 = jnp.sum(jnp.where(oh1, pos_x, 0.0), axis=-1, keepdims=True)
    p2 = jnp.sum(jnp.where(oh2, pos_x, 0.0), axis=-1, keepdims=True)
    pos_ref[...] = jnp.concatenate([p1, p2], axis=1).astype(jnp.int32)
    new_cnt = cnt + cum[BG - 1:BG, :]
    cnt_ref[...] = new_cnt
    counts_ref[...] = new_cnt.astype(jnp.int32)


def _gate(x, gate_W, gate_b):
    grid = TOK // BG
    out_shapes = (
        jax.ShapeDtypeStruct((TOK, TOPK), jnp.int32),
        jax.ShapeDtypeStruct((TOK, TOPK), jnp.float32),
        jax.ShapeDtypeStruct((TOK, TOPK), jnp.int32),
        jax.ShapeDtypeStruct((1, E), jnp.int32),
    )
    return pl.pallas_call(
        _gate_body,
        grid=(grid,),
        in_specs=[
            pl.BlockSpec((BG, D), lambda i: (i, 0)),
            pl.BlockSpec((D, E), lambda i: (0, 0)),
            pl.BlockSpec((1, E), lambda i: (0, 0)),
        ],
        out_specs=(
            pl.BlockSpec((BG, TOPK), lambda i: (i, 0)),
            pl.BlockSpec((BG, TOPK), lambda i: (i, 0)),
            pl.BlockSpec((BG, TOPK), lambda i: (i, 0)),
            pl.BlockSpec((1, E), lambda i: (0, 0)),
        ),
        out_shape=out_shapes,
        scratch_shapes=[pltpu.VMEM((1, E), jnp.float32)],
    )(x, gate_W, gate_b.reshape(1, E))


NCHG = ECAP // CH
PERW = (NCHG + NW - 1) // NW
WIN = 16

_SC_SCRATCH = [
    pltpu.VMEM((E, WIN, CH), jnp.int32),
    pltpu.VMEM((CH, D), jnp.float32),
    pltpu.VMEM((CH, D), jnp.float32),
    pltpu.VMEM((16,), jnp.int32),
    pltpu.SemaphoreType.DMA,
    pltpu.SemaphoreType.DMA,
    pltpu.SemaphoreType.DMA,
    pltpu.SemaphoreType.DMA,
    pltpu.SemaphoreType.DMA,
]


def _sc_copy_loop(wid, nch_vec, idx_win, idx2_hbm, scr, mkA, mkB):
    rows0, rows1 = scr[1], scr[2]
    wsem = scr[4]
    sa0, sa1, sb0, sb1 = scr[5], scr[6], scr[7], scr[8]
    bufs = ((rows0, sa0, sb0), (rows1, sa1, sb1))
    pers = []
    los = []
    offs = []
    for e in range(E):
        n = nch_vec[e]
        per = (n + NW - 1) // NW
        lo = jnp.minimum(wid * per, n)
        s = ((e * NCHG + lo) // 8) * 8
        pers.append(per)
        los.append(lo)
        offs.append(e * NCHG + lo - s)
        pltpu.async_copy(idx2_hbm.at[pl.ds(s, WIN)], idx_win.at[e], wsem)
    for e in range(E):
        pltpu.make_async_copy(idx2_hbm.at[pl.ds(0, WIN)],
                              idx_win.at[e], wsem).wait()
    for e in range(E):
        n = nch_vec[e]
        lo = los[e]
        off = offs[e]
        cnt = jnp.maximum(jnp.minimum(pers[e], n - lo), 0)

        def body(k2, _, e=e, lo=lo, off=off, cnt=cnt):
            for b in range(2):
                k = 2 * k2 + b
                rowsv, semA, semB = bufs[b]
                rowsv_p, semA_p, semB_p = bufs[1 - b]

                @pl.when(k < cnt)
                def _(k=k, rowsv=rowsv, semA=semA, semB=semB):
                    @pl.when(k >= 2)
                    def _():
                        mkB(e, off + k - 2, lo + k - 2,
                            rowsv, semB, False).wait()

                    mkA(e, off + k, lo + k, rowsv, semA, True)

                @pl.when((k >= 1) & (k <= cnt))
                def _(k=k, rowsv_p=rowsv_p, semA_p=semA_p, semB_p=semB_p):
                    mkA(e, off + k - 1, lo + k - 1,
                        rowsv_p, semA_p, False).wait()
                    mkB(e, off + k - 1, lo + k - 1, rowsv_p, semB_p, True)
            return 0

        lax.fori_loop(0, (cnt + 2) // 2, body, 0)
        for b in range(2):
            rowsv, _, semB = bufs[b]

            @pl.when(cnt >= b + 1)
            def _(e=e, b=b, off=off, rowsv=rowsv, semB=semB):
                mkB(e, off, b, rowsv, semB, False).wait()


def _sc_gather(x, sorted_tok2, nchunk):
    mesh = plsc.VectorSubcoreMesh(core_axis_name="c", subcore_axis_name="s")

    @functools.partial(
        pl.kernel, mesh=mesh,
        out_type=tuple(jax.ShapeDtypeStruct((ECAP, D), jnp.float32)
                       for _ in range(E)),
        scratch_types=list(_SC_SCRATCH),
    )
    def k(x_hbm, st_hbm, nch_hbm, *rest):
        xg_hbms = rest[:E]
        scr = rest[E:]
        idx_win, nch_v = scr[0], scr[3]
        wid = lax.axis_index("s") * 2 + lax.axis_index("c")
        pltpu.sync_copy(nch_hbm, nch_v)
        nch_vec = nch_v[...]

        def mkA(e, k, c, rowsv, sem, start):
            cp = (pltpu.async_copy if start else pltpu.make_async_copy)
            return cp(x_hbm.at[idx_win.at[e, k]], rowsv, sem)

        def mkB(e, k, c, rowsv, sem, start):
            cp = (pltpu.async_copy if start else pltpu.make_async_copy)
            return cp(rowsv, xg_hbms[e].at[pl.ds(c * CH, CH)], sem)

        _sc_copy_loop(wid, nch_vec, idx_win, st_hbm, scr, mkA, mkB)

    return k(x, sorted_tok2, nchunk)


def _sc_scatter(ygs, dest2, nchunk):
    mesh = plsc.VectorSubcoreMesh(core_axis_name="c", subcore_axis_name="s")

    @functools.partial(
        pl.kernel, mesh=mesh,
        out_type=jax.ShapeDtypeStruct((TOPK * TOK + CH, D), jnp.float32),
        scratch_types=list(_SC_SCRATCH),
    )
    def k(*args):
        yg_hbms = args[:E]
        dest_hbm, nch_hbm, yc_hbm = args[E:E + 3]
        scr = args[E + 3:]
        idx_win, nch_v = scr[0], scr[3]
        wid = lax.axis_index("s") * 2 + lax.axis_index("c")
        pltpu.sync_copy(nch_hbm, nch_v)
        nch_vec = nch_v[...]

        def mkA(e, k, c, rowsv, sem, start):
            cp = (pltpu.async_copy if start else pltpu.make_async_copy)
            return cp(yg_hbms[e].at[pl.ds(c * CH, CH)], rowsv, sem)

        def mkB(e, k, c, rowsv, sem, start):
            cp = (pltpu.async_copy if start else pltpu.make_async_copy)
            return cp(rowsv, yc_hbm.at[idx_win.at[e, k]], sem)

        _sc_copy_loop(wid, nch_vec, idx_win, dest_hbm, scr, mkA, mkB)

    return k(*ygs, dest2, nchunk)


def _mlp_body(act, depth, nrefs, s_ref, *refs):
    out_ref = refs[-1]
    i = pl.program_id(0)

    @pl.when(i < s_ref[0])
    def _():
        h = refs[0][...]
        for j in range(depth):
            W, b, g, beta = refs[1 + 4 * j: 5 + 4 * j]
            h = jnp.dot(h, W[...], preferred_element_type=jnp.float32) + b[...]
            h = _apply_act(act, h)
            h = _layernorm(h, g[...], beta[...])
        Wo, bo, go, betao = refs[1 + 4 * depth: 5 + 4 * depth]
        h = jnp.dot(h, Wo[...], preferred_element_type=jnp.float32) + bo[...]
        h = _layernorm(h, go[...], betao[...])
        out_ref[...] = h


def _expert_sparse(i, ep, xg_e, nblk_e):
    act, depth, hid = _cfg_k(i)

    def xmap(b, s):
        j = jnp.maximum(jnp.minimum(b, s[0] - 1), 0)
        return (j, 0)

    def cmap(b, s):
        return (0, 0)

    args = [nblk_e, xg_e]
    in_specs = [pl.BlockSpec((BT, D), xmap)]
    in_dim = D
    for lyr in ep["layers"]:
        args += [lyr["W"], lyr["b"].reshape(1, hid), lyr["g"].reshape(1, hid),
                 lyr["beta"].reshape(1, hid)]
        in_specs += [
            pl.BlockSpec((in_dim, hid), cmap),
            pl.BlockSpec((1, hid), cmap),
            pl.BlockSpec((1, hid), cmap),
            pl.BlockSpec((1, hid), cmap),
        ]
        in_dim = hid
    o = ep["out"]
    args += [o["W"], o["b"].reshape(1, D), o["g"].reshape(1, D),
             o["beta"].reshape(1, D)]
    in_specs += [
        pl.BlockSpec((in_dim, D), cmap),
        pl.BlockSpec((1, D), cmap),
        pl.BlockSpec((1, D), cmap),
        pl.BlockSpec((1, D), cmap),
    ]
    return pl.pallas_call(
        functools.partial(_mlp_body, act, depth, len(in_specs)),
        grid_spec=pltpu.PrefetchScalarGridSpec(
            num_scalar_prefetch=1,
            grid=(NBLK_MAX,),
            in_specs=in_specs,
            out_specs=pl.BlockSpec((BT, D), xmap),
        ),
        out_shape=jax.ShapeDtypeStruct((ECAP, D), jnp.float32),
    )(*args)


def _add_body(w_ref, a_ref, b_ref, o_ref):
    w = w_ref[...]
    o_ref[...] = a_ref[...] * w[:, 0:1] + b_ref[...] * w[:, 1:2]


def _final_add(wtop, ycat):
    grid = TOK // BT
    return pl.pallas_call(
        _add_body,
        grid=(grid,),
        in_specs=[
            pl.BlockSpec((BT, TOPK), lambda i: (i, 0)),
            pl.BlockSpec((BT, D), lambda i: (i, 0)),
            pl.BlockSpec((BT, D), lambda i: (TOK // BT + i, 0)),
        ],
        out_specs=pl.BlockSpec((BT, D), lambda i: (i, 0)),
        out_shape=jax.ShapeDtypeStruct((TOK, D), jnp.float32),
    )(wtop, ycat, ycat)


def kernel(x, experts, gate_W, gate_b):
    idx, wtop, pos, counts = _gate(x, gate_W, gate_b)

    tok = jnp.arange(TOK, dtype=jnp.int32)
    slot = idx * ECAP + pos
    sflat = jnp.concatenate([slot[:, 0], slot[:, 1]])
    dest = (jnp.full((E * ECAP,), TOPK * TOK, jnp.int32)
            .at[sflat].set(jnp.concatenate([tok, tok + TOK]),
                           unique_indices=True))
    sorted_tok = dest & (TOK - 1)
    nblk = (counts[0] + BT - 1) // BT
    nchunk16 = jnp.pad(nblk * CPB, (0, 16 - E))

    st2 = jnp.concatenate(
        [sorted_tok.reshape(E * NCHG, CH),
         jnp.zeros((WIN, CH), jnp.int32)], axis=0)
    xgs = _sc_gather(x, st2, nchunk16)
    ygs = [
        _expert_sparse(i, ep, xgs[i], nblk[i:i + 1])
        for i, ep in enumerate(experts)
    ]
    dest2 = jnp.concatenate(
        [dest.reshape(E * NCHG, CH),
         jnp.full((WIN, CH), TOPK * TOK, jnp.int32)], axis=0)
    ycat = _sc_scatter(ygs, dest2, nchunk16)
    return _final_add(wtop, ycat)
